# baseline (speedup 1.0000x reference)
import jax
import jax.numpy as jnp
from jax import lax
from jax.experimental import pallas as pl
from jax.experimental.pallas import tpu as pltpu
from jax.experimental.pallas import tpu_sc as plsc

_B, _S = 1024, 50
_N = _B * _S
_VOCAB = 100000
_DC = 32
_DP = 32
_DB = 64
_NA = 8
_NF = 8
_FD = 32
_RA = _NA * _DB

_NC, _NS = 2, 16
_NW = _NC * _NS
_CH = 128
_NCH = _B // _CH
_L = 16


def _sc_gather_body(idx_hbm, enc_hbm, lwt_hbm, enc_out, lw_out,
                    idx_v, rows_v, lwT_v, sem):
    wid = lax.axis_index("s") * _NC + lax.axis_index("c")

    def do_panel(p):
        pltpu.sync_copy(idx_hbm.at[p], idx_v)
        copies = []
        for j in range(_NCH):
            copies.append(pltpu.async_copy(
                enc_hbm.at[idx_v.at[j]], rows_v.at[pl.ds(j * _CH, _CH)], sem))
        def lw_chunk(c, acc):
            for f in range(_NA):
                pltpu.async_copy(
                    lwt_hbm.at[f].at[idx_v.at[c]],
                    lwT_v.at[f, pl.ds(c * _CH, _CH)], sem)
            return acc
        lax.fori_loop(0, _NCH, lw_chunk, jnp.int32(0))
        for c in copies:
            c.wait()
        pltpu.make_async_copy(
            lwt_hbm.at[:, pl.ds(0, _B)], lwT_v, sem).wait()

        pltpu.sync_copy(rows_v, enc_out.at[pl.ds(p * _B, _B)])
        pltpu.sync_copy(lwT_v, lw_out.at[p])

    for c in range(2):
        p = wid + _NW * c
        @pl.when(p < _S)
        def _():
            do_panel(p)


@jax.jit
def _sc_gather(idx3, enc_table, lwt_table):
    mesh = plsc.VectorSubcoreMesh(core_axis_name="c", subcore_axis_name="s")
    return pl.kernel(
        _sc_gather_body,
        out_type=(
            jax.ShapeDtypeStruct((_N, _DC), jnp.float32),
            jax.ShapeDtypeStruct((_S, _NA, _B), jnp.float32),
        ),
        mesh=mesh,
        scratch_types=[
            pltpu.VMEM((_NCH, _CH), jnp.int32),
            pltpu.VMEM((_B, _DC), jnp.float32),
            pltpu.VMEM((_NA, _B), jnp.float32),
            pltpu.SemaphoreType.DMA,
        ],
        compiler_params=pltpu.CompilerParams(use_tc_tiling_on_sc=False),
    )(idx3, enc_table, lwt_table)


def _polysin(ang):
    n = jnp.round(ang * jnp.float32(1.0 / jnp.pi))
    r = ang - n * jnp.float32(jnp.pi)
    r2 = r * r
    p = jnp.float32(2.755732e-6) + r2 * jnp.float32(-2.5052108e-8)
    p = jnp.float32(-1.9841270e-4) + r2 * p
    p = jnp.float32(8.3333331e-3) + r2 * p
    p = jnp.float32(-1.6666667e-1) + r2 * p
    s = r * (jnp.float32(1.0) + r2 * p)
    sgn = jnp.left_shift(n.astype(jnp.int32), 31)
    return lax.bitcast_convert_type(
        lax.bitcast_convert_type(s, jnp.int32) ^ sgn, jnp.float32)


def _tc_body(g_ref, c_ref, xy_ref, k_ref, w3_ref, aoff_ref, sp_ref, out_ref):
    g4T = jnp.transpose(g_ref[...])
    gT = jnp.dot(g4T[0:32, :], sp_ref[0], preferred_element_type=jnp.float32)
    for q in range(1, 4):
        gT = gT + jnp.dot(g4T[32 * q:32 * q + 32, :], sp_ref[q],
                          preferred_element_type=jnp.float32)
    c = c_ref[0]
    for t in range(3):
        m = jnp.broadcast_to(c == t, (_DC, _B))
        sp = jnp.broadcast_to(k_ref[:, t + 3:t + 4], (_DC, _B))
        gT = jnp.where(m, sp, gT)
    x = jnp.broadcast_to(xy_ref[0, 0:1, :], (_FD, _B))
    y = jnp.broadcast_to(xy_ref[0, 1:2, :], (_FD, _B))
    half = jnp.broadcast_to(k_ref[:, 2:3] != 0, (_FD, _B))
    ang = jnp.where(half, x, y) * jnp.broadcast_to(k_ref[:, 0:1], (_FD, _B)) \
        + jnp.broadcast_to(k_ref[:, 1:2], (_FD, _B))
    fourT = _polysin(ang)
    gfT = jnp.concatenate([gT, fourT], axis=0)
    o = jnp.dot(w3_ref[...], gfT, preferred_element_type=jnp.float32)
    out_ref[0] = o + jnp.broadcast_to(aoff_ref[...], (_RA, _B))


@jax.jit
def _tc_dense(gT3, c3, xy3, k, w3, aoff, spread):
    return pl.pallas_call(
        _tc_body,
        grid=(_S,),
        in_specs=[
            pl.BlockSpec((_B * _DC // 128, 128), lambda i: (i, 0)),
            pl.BlockSpec((1, 1, _B), lambda i: (i, 0, 0)),
            pl.BlockSpec((1, 2, _B), lambda i: (i, 0, 0)),
            pl.BlockSpec((_DC, 8), lambda i: (0, 0)),
            pl.BlockSpec((_RA, _DB), lambda i: (0, 0)),
            pl.BlockSpec((_RA, 1), lambda i: (0, 0)),
            pl.BlockSpec((4, _B // 4, _B), lambda i: (0, 0, 0)),
        ],
        out_specs=pl.BlockSpec((1, _RA, _B), lambda i: (i, 0, 0)),
        out_shape=jax.ShapeDtypeStruct((_S, _RA, _B), jnp.float32),
    )(gT3, c3, xy3, k, w3, aoff, spread)


def kernel(content, coords, content_encoding, special_pad, special_eos,
           special_empty, freqs, proj_W, atom_offsets, log_weights):
    content_T = content.T
    idx3 = content_T.reshape(_S, _NCH, _CH)
    encT3, lwT3 = _sc_gather(idx3, content_encoding, log_weights.T)

    f = freqs.astype(jnp.float32)
    rep2 = jnp.repeat(f, 2)
    frow = jnp.concatenate([rep2, rep2])
    phase = jnp.tile(jnp.array([0.0, 1.0], jnp.float32), _FD // 2) \
        * jnp.float32(jnp.pi / 2)
    half = (jnp.arange(_FD) < 16).astype(jnp.float32)
    k = jnp.stack([frow, phase, half, special_pad, special_eos,
                   special_empty, jnp.zeros((_FD,), jnp.float32),
                   jnp.zeros((_FD,), jnp.float32)], axis=1)
    blkT = jnp.zeros((_DB, _DB), jnp.float32)
    blkT = blkT.at[:_DC, :_DC].set(jnp.eye(_DC, dtype=jnp.float32))
    blkT = blkT.at[_DC:, _DC:].set(proj_W.astype(jnp.float32))
    w3 = jnp.tile(blkT, (_NA, 1))
    aoffT = atom_offsets.reshape(_RA, 1)

    rr = jnp.arange(_B // 4, dtype=jnp.int32)
    cc = jnp.arange(_B, dtype=jnp.int32)
    qq = jnp.arange(4, dtype=jnp.int32)
    spread = (cc[None, None, :] ==
              4 * rr[None, :, None] + qq[:, None, None]).astype(jnp.float32)

    c3 = content_T.reshape(_S, 1, _B)
    xy3 = coords.transpose(1, 2, 0)
    enc_packed = encT3.reshape(_N * _DC // 128, 128)
    out3 = _tc_dense(enc_packed, c3, xy3, k, w3, aoffT, spread)
    atoms = out3.reshape(_S, _NA, _DB, _B).transpose(3, 0, 1, 2)
    return atoms, lwT3.transpose(2, 0, 1)


# speedup vs baseline: 1.0014x; 1.0014x over previous
import jax
import jax.numpy as jnp
from jax import lax
from jax.experimental import pallas as pl
from jax.experimental.pallas import tpu as pltpu
from jax.experimental.pallas import tpu_sc as plsc

_B, _S = 1024, 50
_N = _B * _S
_VOCAB = 100000
_DC = 32
_DP = 32
_DB = 64
_NA = 8
_NF = 8
_FD = 32
_RA = _NA * _DB

_NC, _NS = 2, 16
_NW = _NC * _NS
_CH = 128
_NCH = _B // _CH
_L = 16


def _sc_gather_body(idx_hbm, enc_hbm, lwt_hbm, enc_out, lw_out,
                    idx_v, rows_v, lwT_v, sem):
    wid = lax.axis_index("s") * _NC + lax.axis_index("c")

    def do_panel(p):
        pltpu.sync_copy(idx_hbm.at[p], idx_v)
        copies = []
        for j in range(_NCH):
            copies.append(pltpu.async_copy(
                enc_hbm.at[idx_v.at[j]], rows_v.at[pl.ds(j * _CH, _CH)], sem))
        def lw_chunk(c, acc):
            for f in range(_NA):
                pltpu.async_copy(
                    lwt_hbm.at[f].at[idx_v.at[c]],
                    lwT_v.at[f, pl.ds(c * _CH, _CH)], sem)
            return acc
        lax.fori_loop(0, _NCH, lw_chunk, jnp.int32(0))
        for c in copies:
            c.wait()
        pltpu.make_async_copy(
            lwt_hbm.at[:, pl.ds(0, _B)], lwT_v, sem).wait()

        pltpu.sync_copy(rows_v, enc_out.at[pl.ds(p * _B, _B)])
        pltpu.sync_copy(lwT_v, lw_out.at[p])

    for c in range(2):
        p = wid + _NW * c
        @pl.when(p < _S)
        def _():
            do_panel(p)


@jax.jit
def _sc_gather(idx3, enc_table, lwt_table):
    mesh = plsc.VectorSubcoreMesh(core_axis_name="c", subcore_axis_name="s")
    return pl.kernel(
        _sc_gather_body,
        out_type=(
            jax.ShapeDtypeStruct((_N, _DC), jnp.float32),
            jax.ShapeDtypeStruct((_S, _NA, _B), jnp.float32),
        ),
        mesh=mesh,
        scratch_types=[
            pltpu.VMEM((_NCH, _CH), jnp.int32),
            pltpu.VMEM((_B, _DC), jnp.float32),
            pltpu.VMEM((_NA, _B), jnp.float32),
            pltpu.SemaphoreType.DMA,
        ],
        compiler_params=pltpu.CompilerParams(use_tc_tiling_on_sc=False),
    )(idx3, enc_table, lwt_table)


def _polysin(ang):
    n = jnp.round(ang * jnp.float32(1.0 / jnp.pi))
    r = ang - n * jnp.float32(jnp.pi)
    r2 = r * r
    p = jnp.float32(2.755732e-6) + r2 * jnp.float32(-2.5052108e-8)
    p = jnp.float32(-1.9841270e-4) + r2 * p
    p = jnp.float32(8.3333331e-3) + r2 * p
    p = jnp.float32(-1.6666667e-1) + r2 * p
    s = r * (jnp.float32(1.0) + r2 * p)
    sgn = jnp.left_shift(n.astype(jnp.int32), 31)
    return lax.bitcast_convert_type(
        lax.bitcast_convert_type(s, jnp.int32) ^ sgn, jnp.float32)


def _tc_body(g_ref, c_ref, xy_ref, k_ref, w3_ref, aoff_ref, sp_ref, out_ref):
    g4T = jnp.transpose(g_ref[...])
    gT = jnp.dot(g4T[0:32, :], sp_ref[0], preferred_element_type=jnp.float32)
    for q in range(1, 4):
        gT = gT + jnp.dot(g4T[32 * q:32 * q + 32, :], sp_ref[q],
                          preferred_element_type=jnp.float32)
    c = c_ref[0]
    for t in range(3):
        m = jnp.broadcast_to(c == t, (_DC, _B))
        sp = jnp.broadcast_to(k_ref[:, t + 3:t + 4], (_DC, _B))
        gT = jnp.where(m, sp, gT)
    x = jnp.broadcast_to(xy_ref[0, 0:1, :], (_FD, _B))
    y = jnp.broadcast_to(xy_ref[0, 1:2, :], (_FD, _B))
    half = jnp.broadcast_to(k_ref[:, 2:3] != 0, (_FD, _B))
    ang = jnp.where(half, x, y) * jnp.broadcast_to(k_ref[:, 0:1], (_FD, _B)) \
        + jnp.broadcast_to(k_ref[:, 1:2], (_FD, _B))
    fourT = _polysin(ang)
    gfT = jnp.concatenate([gT, fourT], axis=0)
    o = jnp.dot(w3_ref[...], gfT, preferred_element_type=jnp.float32)
    out_ref[0] = o + jnp.broadcast_to(aoff_ref[...], (_RA, _B))


@jax.jit
def _tc_dense(gT3, c3, xy3, k, w3, aoff, spread):
    return pl.pallas_call(
        _tc_body,
        grid=(_S,),
        in_specs=[
            pl.BlockSpec((_B * _DC // 128, 128), lambda i: (i, 0)),
            pl.BlockSpec((1, 1, _B), lambda i: (i, 0, 0)),
            pl.BlockSpec((1, 2, _B), lambda i: (i, 0, 0)),
            pl.BlockSpec((_DC, 8), lambda i: (0, 0)),
            pl.BlockSpec((_RA, _DB), lambda i: (0, 0)),
            pl.BlockSpec((_RA, 1), lambda i: (0, 0)),
            pl.BlockSpec((4, _B // 4, _B), lambda i: (0, 0, 0)),
        ],
        out_specs=pl.BlockSpec((1, _RA, _B), lambda i: (i, 0, 0)),
        out_shape=jax.ShapeDtypeStruct((_S, _RA, _B), jnp.float32),
        compiler_params=pltpu.CompilerParams(
            dimension_semantics=("parallel",)),
    )(gT3, c3, xy3, k, w3, aoff, spread)


def kernel(content, coords, content_encoding, special_pad, special_eos,
           special_empty, freqs, proj_W, atom_offsets, log_weights):
    content_T = content.T
    idx3 = content_T.reshape(_S, _NCH, _CH)
    encT3, lwT3 = _sc_gather(idx3, content_encoding, log_weights.T)

    f = freqs.astype(jnp.float32)
    rep2 = jnp.repeat(f, 2)
    frow = jnp.concatenate([rep2, rep2])
    phase = jnp.tile(jnp.array([0.0, 1.0], jnp.float32), _FD // 2) \
        * jnp.float32(jnp.pi / 2)
    half = (jnp.arange(_FD) < 16).astype(jnp.float32)
    k = jnp.stack([frow, phase, half, special_pad, special_eos,
                   special_empty, jnp.zeros((_FD,), jnp.float32),
                   jnp.zeros((_FD,), jnp.float32)], axis=1)
    blkT = jnp.zeros((_DB, _DB), jnp.float32)
    blkT = blkT.at[:_DC, :_DC].set(jnp.eye(_DC, dtype=jnp.float32))
    blkT = blkT.at[_DC:, _DC:].set(proj_W.astype(jnp.float32))
    w3 = jnp.tile(blkT, (_NA, 1))
    aoffT = atom_offsets.reshape(_RA, 1)

    rr = jnp.arange(_B // 4, dtype=jnp.int32)
    cc = jnp.arange(_B, dtype=jnp.int32)
    qq = jnp.arange(4, dtype=jnp.int32)
    spread = (cc[None, None, :] ==
              4 * rr[None, :, None] + qq[:, None, None]).astype(jnp.float32)

    c3 = content_T.reshape(_S, 1, _B)
    xy3 = coords.transpose(1, 2, 0)
    enc_packed = encT3.reshape(_N * _DC // 128, 128)
    out3 = _tc_dense(enc_packed, c3, xy3, k, w3, aoffT, spread)
    atoms = out3.reshape(_S, _NA, _DB, _B).transpose(3, 0, 1, 2)
    return atoms, lwT3.transpose(2, 0, 1)
